# baseline (speedup 1.0000x reference)
import functools

import jax
import jax.numpy as jnp
from jax import lax
from jax.experimental import pallas as pl
from jax.experimental.pallas import tpu as pltpu
from jax.experimental.pallas import tpu_sc as plsc

_DIM = 64
_NC = 2
_NS = 16
_NW = _NC * _NS

_CHUNK = 128
_IDX_PER_STREAM = 128
_STREAMS = _CHUNK // _IDX_PER_STREAM
_LANES = 16
_ROW_UNROLL = 8


def _pe_concat_kernel(n_iters, x_ref, t_ref, pe2_ref, out_ref,
                      idx_v, outv, xv, sem_idx, sem_x, sem_g, sem_w):
    wid = lax.axis_index("s") * _NC + lax.axis_index("c")
    start = wid * n_iters * _CHUNK

    pltpu.async_copy(t_ref.at[pl.ds(start, _CHUNK)], idx_v.at[0], sem_idx)
    pltpu.async_copy(x_ref.at[pl.ds(start, _CHUNK)], xv.at[0], sem_x)

    def body(i, _):
        r = lax.rem(i, 2)
        base = start + i * _CHUNK
        pltpu.make_async_copy(t_ref.at[pl.ds(base, _CHUNK)],
                              idx_v.at[r], sem_idx).wait()
        gathers = []
        for j in range(_STREAMS):
            gathers.append(pltpu.async_copy(
                pe2_ref.at[idx_v.at[r, pl.ds(j * _IDX_PER_STREAM,
                                             _IDX_PER_STREAM)]],
                outv.at[r, pl.ds(j * _IDX_PER_STREAM, _IDX_PER_STREAM)],
                sem_g))

        @pl.when(i + 1 < n_iters)
        def _prefetch():
            nbase = base + _CHUNK
            pltpu.async_copy(t_ref.at[pl.ds(nbase, _CHUNK)],
                             idx_v.at[1 - r], sem_idx)
            pltpu.async_copy(x_ref.at[pl.ds(nbase, _CHUNK)],
                             xv.at[1 - r], sem_x)

        pltpu.make_async_copy(x_ref.at[pl.ds(base, _CHUNK)],
                              xv.at[r], sem_x).wait()
        for g in gathers:
            g.wait()

        def vbody(k, _):
            r0 = k * _ROW_UNROLL
            for u in range(_ROW_UNROLL):
                for c in range(_DIM // _LANES):
                    outv[r, r0 + u, pl.ds(c * _LANES, _LANES)] = (
                        xv[r, r0 + u, pl.ds(c * _LANES, _LANES)])
            return ()

        lax.fori_loop(0, _CHUNK // _ROW_UNROLL, vbody, ())

        @pl.when(i >= 1)
        def _drain():
            pltpu.make_async_copy(
                outv.at[1 - r],
                out_ref.at[pl.ds(base - _CHUNK, _CHUNK)], sem_w).wait()

        pltpu.async_copy(outv.at[r], out_ref.at[pl.ds(base, _CHUNK)], sem_w)
        return ()

    lax.fori_loop(0, n_iters, body, ())

    last_r = lax.rem(n_iters - 1, 2)
    last_base = start + (n_iters - 1) * _CHUNK
    pltpu.make_async_copy(outv.at[last_r],
                          out_ref.at[pl.ds(last_base, _CHUNK)], sem_w).wait()


def kernel(x, t, pe):
    batch, seq, dim = x.shape
    n = batch * seq
    assert n % (_NW * _CHUNK) == 0
    n_iters = n // (_NW * _CHUNK)

    x2 = x.reshape(n, dim)
    t1 = t.reshape(n)
    pe2 = jnp.concatenate([jnp.zeros_like(pe), pe], axis=1)

    mesh = plsc.VectorSubcoreMesh(core_axis_name="c", subcore_axis_name="s")
    out = pl.kernel(
        functools.partial(_pe_concat_kernel, n_iters),
        out_type=jax.ShapeDtypeStruct((n, 2 * dim), jnp.float32),
        mesh=mesh,
        scratch_types=[
            pltpu.VMEM((2, _CHUNK), jnp.int32),
            pltpu.VMEM((2, _CHUNK, 2 * dim), jnp.float32),
            pltpu.VMEM((2, _CHUNK, dim), jnp.float32),
            pltpu.SemaphoreType.DMA,
            pltpu.SemaphoreType.DMA,
            pltpu.SemaphoreType.DMA,
            pltpu.SemaphoreType.DMA,
        ],
    )(x2, t1, pe2)
    return out.reshape(batch, seq, 2 * dim)


# speedup vs baseline: 1.0353x; 1.0353x over previous
import functools

import jax
import jax.numpy as jnp
from jax import lax
from jax.experimental import pallas as pl
from jax.experimental.pallas import tpu as pltpu
from jax.experimental.pallas import tpu_sc as plsc

_DIM = 64
_NC = 2
_NS = 16
_NW = _NC * _NS

_CHUNK = 256
_IDX_PER_STREAM = 128
_STREAMS = _CHUNK // _IDX_PER_STREAM
_LANES = 16
_ROW_UNROLL = 8


def _pe_concat_kernel(n_iters, x_ref, t_ref, pe2_ref, out_ref,
                      idx_v, outv, xv, sem, sem_w):
    wid = lax.axis_index("s") * _NC + lax.axis_index("c")
    start = wid * n_iters * _CHUNK

    def body(i, _):
        r = lax.rem(i, 2)
        base = start + i * _CHUNK
        pltpu.sync_copy(t_ref.at[pl.ds(base, _CHUNK)], idx_v)
        copies = []
        for j in range(_STREAMS):
            copies.append(pltpu.async_copy(
                pe2_ref.at[idx_v.at[pl.ds(j * _IDX_PER_STREAM,
                                          _IDX_PER_STREAM)]],
                outv.at[r, pl.ds(j * _IDX_PER_STREAM, _IDX_PER_STREAM)],
                sem))
        copies.append(pltpu.async_copy(
            x_ref.at[pl.ds(base, _CHUNK)], xv, sem))
        for c in copies:
            c.wait()

        def vbody(k, _):
            r0 = k * _ROW_UNROLL
            for u in range(_ROW_UNROLL):
                for c in range(_DIM // _LANES):
                    outv[r, r0 + u, pl.ds(c * _LANES, _LANES)] = (
                        xv[r0 + u, pl.ds(c * _LANES, _LANES)])
            return ()

        lax.fori_loop(0, _CHUNK // _ROW_UNROLL, vbody, ())

        @pl.when(i >= 1)
        def _drain():
            pltpu.make_async_copy(
                outv.at[1 - r],
                out_ref.at[pl.ds(base - _CHUNK, _CHUNK)], sem_w).wait()

        pltpu.async_copy(outv.at[r], out_ref.at[pl.ds(base, _CHUNK)], sem_w)
        return ()

    lax.fori_loop(0, n_iters, body, ())

    last_r = lax.rem(n_iters - 1, 2)
    last_base = start + (n_iters - 1) * _CHUNK
    pltpu.make_async_copy(outv.at[last_r],
                          out_ref.at[pl.ds(last_base, _CHUNK)], sem_w).wait()


def kernel(x, t, pe):
    batch, seq, dim = x.shape
    n = batch * seq
    assert n % (_NW * _CHUNK) == 0
    n_iters = n // (_NW * _CHUNK)

    x2 = x.reshape(n, dim)
    t1 = t.reshape(n)
    pe2 = jnp.concatenate([jnp.zeros_like(pe), pe], axis=1)

    mesh = plsc.VectorSubcoreMesh(core_axis_name="c", subcore_axis_name="s")
    out = pl.kernel(
        functools.partial(_pe_concat_kernel, n_iters),
        out_type=jax.ShapeDtypeStruct((n, 2 * dim), jnp.float32),
        mesh=mesh,
        scratch_types=[
            pltpu.VMEM((_CHUNK,), jnp.int32),
            pltpu.VMEM((2, _CHUNK, 2 * dim), jnp.float32),
            pltpu.VMEM((_CHUNK, dim), jnp.float32),
            pltpu.SemaphoreType.DMA,
            pltpu.SemaphoreType.DMA,
        ],
    )(x2, t1, pe2)
    return out.reshape(batch, seq, 2 * dim)


# speedup vs baseline: 1.0854x; 1.0484x over previous
import functools

import jax
import jax.numpy as jnp
from jax import lax
from jax.experimental import pallas as pl
from jax.experimental.pallas import tpu as pltpu
from jax.experimental.pallas import tpu_sc as plsc

_DIM = 64
_NC = 2
_NS = 16
_NW = _NC * _NS

_ROWS = 2
_LANES = 16


def _pe_concat_kernel(seq, rows_per_worker, x_ref, t_ref, pe2_ref, out_ref,
                      idx_v, outv, xv, sem):
    wid = lax.axis_index("s") * _NC + lax.axis_index("c")
    row_start = wid * rows_per_worker
    chunk = _ROWS * seq
    n_iters = rows_per_worker // _ROWS

    splits = [(o, min(128, chunk - o)) for o in range(0, chunk, 128)]

    def body(it, _):
        row = row_start + it * _ROWS
        base = row * seq
        pltpu.sync_copy(t_ref.at[pl.ds(base, chunk)], idx_v)
        copies = []
        for off, cnt in splits:
            copies.append(pltpu.async_copy(
                pe2_ref.at[idx_v.at[pl.ds(off, cnt)]],
                outv.at[pl.ds(off, cnt)],
                sem))
        copies.append(pltpu.async_copy(
            x_ref.at[pl.ds(row, _ROWS)], xv, sem))
        for c in copies:
            c.wait()

        def vbody(s, _):
            for j in range(_ROWS):
                for c in range(_DIM // _LANES):
                    outv[j * seq + s, pl.ds(c * _LANES, _LANES)] = (
                        xv[j, s, pl.ds(c * _LANES, _LANES)])
            return ()

        lax.fori_loop(0, seq, vbody, ())

        pltpu.sync_copy(outv, out_ref.at[pl.ds(base, chunk)])
        return ()

    lax.fori_loop(0, n_iters, body, ())


def kernel(x, t, pe):
    batch, seq, dim = x.shape
    n = batch * seq
    assert batch % (_NW * _ROWS) == 0
    rows_per_worker = batch // _NW

    t1 = t.reshape(n)
    pe2 = jnp.concatenate([jnp.zeros_like(pe), pe], axis=1)

    mesh = plsc.VectorSubcoreMesh(core_axis_name="c", subcore_axis_name="s")
    out = pl.kernel(
        functools.partial(_pe_concat_kernel, seq, rows_per_worker),
        out_type=jax.ShapeDtypeStruct((n, 2 * dim), jnp.float32),
        mesh=mesh,
        scratch_types=[
            pltpu.VMEM((_ROWS * seq,), jnp.int32),
            pltpu.VMEM((_ROWS * seq, 2 * dim), jnp.float32),
            pltpu.VMEM((_ROWS, seq, dim), jnp.float32),
            pltpu.SemaphoreType.DMA,
        ],
    )(x, t1, pe2)
    return out.reshape(batch, seq, 2 * dim)
